# Initial kernel scaffold; baseline (speedup 1.0000x reference)
import jax
import jax.numpy as jnp
from jax.experimental import pallas as pl


def kernel(embeddings, edges, W, b):
    raise NotImplementedError("write your pallas kernel here")


# speedup vs baseline: 1.1575x; 1.1575x over previous
import functools

import jax
import jax.numpy as jnp
from jax import lax
from jax.experimental import pallas as pl
from jax.experimental.pallas import tpu as pltpu
from jax.experimental.pallas import tpu_sc as plsc

N_NODES = 10000
N_EDGES = 320000
D = 128

NC = 2
NS = 16
L = 16
NW = NC * NS
EPW = N_EDGES // NW
C = 80
M = EPW // C
G = C // L


def _prescale_body(emb_ref, w_ref, out_ref):
    out_ref[...] = emb_ref[...] * w_ref[...]


def _prescale(embeddings, W):
    return pl.pallas_call(
        _prescale_body,
        out_shape=jax.ShapeDtypeStruct((N_NODES, D), jnp.float32),
    )(embeddings, W)


_mesh = plsc.VectorSubcoreMesh(core_axis_name="c", subcore_axis_name="s")


@functools.partial(
    pl.kernel,
    out_type=jax.ShapeDtypeStruct((NW, M, C), jnp.float32),
    mesh=_mesh,
    scratch_types=[
        pltpu.VMEM((M, C), jnp.int32),
        pltpu.VMEM((M, C), jnp.int32),
        pltpu.VMEM((M, C), jnp.float32),
        pltpu.VMEM((C, D), jnp.float32),
        pltpu.VMEM((C, D), jnp.float32),
        pltpu.VMEM((L,), jnp.float32),
        pltpu.SemaphoreType.DMA,
        pltpu.SemaphoreType.DMA,
    ],
    compiler_params=pltpu.CompilerParams(needs_layout_passes=False),
)
def _sc_edge_kernel(we_hbm, emb_hbm, srcidx_hbm, dstidx_hbm, b_hbm, out_hbm,
                    sidx_v, didx_v, out_v, sbuf, dbuf, b_v, sem_s, sem_d):
    wid = lax.axis_index("s") * NC + lax.axis_index("c")
    pltpu.sync_copy(srcidx_hbm.at[wid], sidx_v)
    pltpu.sync_copy(dstidx_hbm.at[wid], didx_v)
    pltpu.sync_copy(b_hbm, b_v)
    bvec = b_v[...]
    lane = lax.iota(jnp.int32, L)

    @pl.loop(0, M)
    def _chunk(c):
        pltpu.async_copy(we_hbm.at[sidx_v.at[c]], sbuf, sem_s)
        pltpu.async_copy(emb_hbm.at[didx_v.at[c]], dbuf, sem_d)
        pltpu.make_async_copy(we_hbm.at[sidx_v.at[c]], sbuf, sem_s).wait()
        pltpu.make_async_copy(emb_hbm.at[didx_v.at[c]], dbuf, sem_d).wait()
        for g in range(G):
            rows = jnp.full((L,), g * L, jnp.int32) + lane
            acc = jnp.zeros((L,), jnp.float32)
            for d in range(D):
                cols = jnp.full((L,), d, jnp.int32)
                s = plsc.load_gather(sbuf, [rows, cols])
                t = plsc.load_gather(dbuf, [rows, cols])
                acc = acc + s * t
            x = acc + bvec
            p = 1.0 / (1.0 + jnp.exp(-x))
            out_v[c, pl.ds(g * L, L)] = p

    pltpu.sync_copy(out_v, out_hbm.at[wid])


def kernel(embeddings, edges, W, b):
    we = _prescale(embeddings, W)
    src = edges[0].reshape(NW, M, C)
    dst = edges[1].reshape(NW, M, C)
    b16 = jnp.full((L,), b[0], jnp.float32)
    out = _sc_edge_kernel(we, embeddings, src, dst, b16)
    return out.reshape(N_EDGES)
